# baseline (speedup 1.0000x reference)
import jax
import jax.numpy as jnp
from jax import lax
from jax.experimental import pallas as pl
from jax.experimental.pallas import tpu as pltpu
from jax.experimental.pallas import tpu_sc as plsc

N = 10000
E = 320000
D = 128
LANES = 16

NUM_CORES = 2
NUM_TILES = 16
NUM_WORKERS = NUM_CORES * NUM_TILES
EPW = E // NUM_WORKERS
K = 80
CHUNKS = EPW // K
ROWS_PER_TILE = 624
OUT_CHUNK = 104
OUT_CHUNKS = ROWS_PER_TILE // OUT_CHUNK
TAIL_BASE = NUM_TILES * ROWS_PER_TILE
TAIL_ROWS = N - TAIL_BASE


def _body(pcr_hbm, val_hbm, feat_hbm, out_hbm,
          acc, ebuf0, ebuf1, ebuf2, ridx0, ridx1, ridx2, val0, val1, val2,
          rows0, rows1, rows2, obuf,
          sem_e0, sem_e1, sem_e2, sem_g0, sem_g1, sem_g2,
          sem_s0, sem_s1, sem_s2):
    cid = lax.axis_index("c")
    sid = lax.axis_index("s")
    wid = cid * NUM_TILES + sid
    edge_base = wid * EPW

    def zrow(r, c):
        for j in range(D // LANES):
            obuf[r, pl.ds(LANES * j, LANES)] = jnp.zeros((LANES,), jnp.float32)
        return c
    lax.fori_loop(0, OUT_CHUNK, zrow, 0)
    row_base = sid * ROWS_PER_TILE
    for c in range(OUT_CHUNKS):
        pltpu.sync_copy(obuf, acc.at[pl.ds(row_base + c * OUT_CHUNK, OUT_CHUNK)])

    @pl.when(sid == NUM_TILES - 1)
    def _():
        pltpu.sync_copy(obuf.at[pl.ds(0, TAIL_ROWS)],
                        acc.at[pl.ds(TAIL_BASE, TAIL_ROWS)])
    plsc.subcore_barrier()

    def e_start(c, S):
        ebuf, ridx_v, val_v, rows_v, sem_e, sem_g, sem_s = S
        base = edge_base + c * K
        pltpu.async_copy(pcr_hbm.at[pl.ds(2 * base, 2 * K)], ebuf, sem_e)
        pltpu.async_copy(val_hbm.at[pl.ds(base, K)], val_v, sem_e)

    def e_wait(c, S):
        ebuf, ridx_v, val_v, rows_v, sem_e, sem_g, sem_s = S
        base = edge_base + c * K
        pltpu.make_async_copy(pcr_hbm.at[pl.ds(2 * base, 2 * K)], ebuf, sem_e).wait()
        pltpu.make_async_copy(val_hbm.at[pl.ds(base, K)], val_v, sem_e).wait()
        for t in range(K // LANES):
            ridx_v[pl.ds(t * LANES, LANES)] = ebuf[pl.ds(K + t * LANES, LANES)]

    def g_start(S):
        ebuf, ridx_v, val_v, rows_v, sem_e, sem_g, sem_s = S
        pltpu.async_copy(feat_hbm.at[ebuf.at[pl.ds(0, K)]], rows_v, sem_g)

    def g_wait(S):
        ebuf, ridx_v, val_v, rows_v, sem_e, sem_g, sem_s = S
        pltpu.make_async_copy(feat_hbm.at[ebuf.at[pl.ds(0, K)]], rows_v, sem_g).wait()

    def s_start(S):
        ebuf, ridx_v, val_v, rows_v, sem_e, sem_g, sem_s = S
        pltpu.async_copy(rows_v, acc.at[ridx_v], sem_s, add=True)

    def s_wait(S):
        ebuf, ridx_v, val_v, rows_v, sem_e, sem_g, sem_s = S
        pltpu.make_async_copy(rows_v, acc.at[ridx_v], sem_s).wait()

    def scale(S):
        ebuf, ridx_v, val_v, rows_v, sem_e, sem_g, sem_s = S

        def e_body(t, cc):
            vv = val_v[pl.ds(t * LANES, LANES)]
            for i in range(LANES):
                e = t * LANES + i
                v = vv[i]
                for j in range(D // LANES):
                    rows_v[e, pl.ds(LANES * j, LANES)] = (
                        rows_v[e, pl.ds(LANES * j, LANES)] * v)
            return cc
        lax.fori_loop(0, K // LANES, e_body, 0)

    sets = [
        (ebuf0, ridx0, val0, rows0, sem_e0, sem_g0, sem_s0),
        (ebuf1, ridx1, val1, rows1, sem_e1, sem_g1, sem_s1),
        (ebuf2, ridx2, val2, rows2, sem_e2, sem_g2, sem_s2),
    ]

    def phase(c, X, Z, drain_prev, prefetch):
        g_wait(X)
        if drain_prev:
            s_wait(Z)
        if prefetch:
            e_start(c + 2, Z)
        scale(X)
        if prefetch:
            e_wait(c + 2, Z)
            g_start(Z)
        s_start(X)

    e_start(0, sets[0])
    e_start(1, sets[1])
    e_wait(0, sets[0])
    g_start(sets[0])
    e_wait(1, sets[1])
    g_start(sets[1])
    phase(0, sets[0], sets[2], False, True)
    phase(1, sets[1], sets[0], True, True)

    def triple_body(p, carry):
        c0 = 3 * p + 2
        phase(c0, sets[2], sets[1], True, True)
        phase(c0 + 1, sets[0], sets[2], True, True)
        phase(c0 + 2, sets[1], sets[0], True, True)
        return carry

    lax.fori_loop(0, (CHUNKS - 5) // 3, triple_body, 0)
    phase(CHUNKS - 3, sets[2], sets[1], True, True)
    phase(CHUNKS - 2, sets[0], sets[2], True, False)
    phase(CHUNKS - 1, sets[1], sets[0], True, False)
    s_wait(sets[1])
    plsc.subcore_barrier()

    for c in range(OUT_CHUNKS):
        sl = pl.ds(row_base + c * OUT_CHUNK, OUT_CHUNK)
        pltpu.sync_copy(acc.at[sl], obuf)
        pltpu.sync_copy(obuf, out_hbm.at[cid].at[sl])

    @pl.when(sid == NUM_TILES - 1)
    def _():
        sl = pl.ds(TAIL_BASE, TAIL_ROWS)
        pltpu.sync_copy(acc.at[sl], obuf.at[pl.ds(0, TAIL_ROWS)])
        pltpu.sync_copy(obuf.at[pl.ds(0, TAIL_ROWS)], out_hbm.at[cid].at[sl])


def _add_body(a_ref, b_ref, o_ref):
    o_ref[...] = a_ref[...] + b_ref[...]


def kernel(adj_indices, adj_values, feature):
    row = adj_indices[0]
    col = adj_indices[1]
    pcr = jnp.stack([col.reshape(NUM_WORKERS * CHUNKS, K),
                     row.reshape(NUM_WORKERS * CHUNKS, K)],
                    axis=1).reshape(2 * E)
    mesh = plsc.VectorSubcoreMesh(
        core_axis_name="c", subcore_axis_name="s", num_cores=NUM_CORES)
    k = pl.kernel(
        _body,
        out_type=jax.ShapeDtypeStruct((NUM_CORES, N, D), jnp.float32),
        mesh=mesh,
        scratch_types=[
            pltpu.VMEM_SHARED((N, D), jnp.float32),
            pltpu.VMEM((2 * K,), jnp.int32),
            pltpu.VMEM((2 * K,), jnp.int32),
            pltpu.VMEM((2 * K,), jnp.int32),
            pltpu.VMEM((K,), jnp.int32),
            pltpu.VMEM((K,), jnp.int32),
            pltpu.VMEM((K,), jnp.int32),
            pltpu.VMEM((K,), jnp.float32),
            pltpu.VMEM((K,), jnp.float32),
            pltpu.VMEM((K,), jnp.float32),
            pltpu.VMEM((K, D), jnp.float32),
            pltpu.VMEM((K, D), jnp.float32),
            pltpu.VMEM((K, D), jnp.float32),
            pltpu.VMEM((OUT_CHUNK, D), jnp.float32),
            pltpu.SemaphoreType.DMA,
            pltpu.SemaphoreType.DMA,
            pltpu.SemaphoreType.DMA,
            pltpu.SemaphoreType.DMA,
            pltpu.SemaphoreType.DMA,
            pltpu.SemaphoreType.DMA,
            pltpu.SemaphoreType.DMA,
            pltpu.SemaphoreType.DMA,
            pltpu.SemaphoreType.DMA,
        ],
    )
    out2 = k(pcr, adj_values, feature)

    blk = 2000
    return pl.pallas_call(
        _add_body,
        out_shape=jax.ShapeDtypeStruct((N, D), jnp.float32),
        grid=(N // blk,),
        in_specs=[pl.BlockSpec((blk, D), lambda i: (i, 0)),
                  pl.BlockSpec((blk, D), lambda i: (i, 0))],
        out_specs=pl.BlockSpec((blk, D), lambda i: (i, 0)),
    )(out2[0], out2[1])


# speedup vs baseline: 1.1041x; 1.1041x over previous
import jax
import jax.numpy as jnp
from jax import lax
from jax.experimental import pallas as pl
from jax.experimental.pallas import tpu as pltpu
from jax.experimental.pallas import tpu_sc as plsc

N = 10000
E = 320000
D = 128
LANES = 16

NUM_CORES = 2
NUM_TILES = 16
NUM_WORKERS = NUM_CORES * NUM_TILES
EPW = E // NUM_WORKERS
K = 80
CHUNKS = EPW // K
ROWS_PER_TILE = 624
OUT_CHUNK = 48
OUT_CHUNKS = ROWS_PER_TILE // OUT_CHUNK
TAIL_BASE = NUM_TILES * ROWS_PER_TILE
TAIL_ROWS = N - TAIL_BASE


def _body(row_hbm, col_hbm, val_hbm, feat_hbm, out_hbm,
          acc, idx0, idx1, idx2, idx3, ridx0, ridx1, ridx2, ridx3,
          val0, val1, val2, val3, rows0, rows1, rows2, rows3, obuf,
          sem_e0, sem_e1, sem_e2, sem_e3, sem_g0, sem_g1, sem_g2, sem_g3,
          sem_s0, sem_s1, sem_s2, sem_s3):
    cid = lax.axis_index("c")
    sid = lax.axis_index("s")
    wid = cid * NUM_TILES + sid
    edge_base = wid * EPW

    def zrow(r, c):
        for j in range(D // LANES):
            obuf[r, pl.ds(LANES * j, LANES)] = jnp.zeros((LANES,), jnp.float32)
        return c
    lax.fori_loop(0, OUT_CHUNK, zrow, 0)
    row_base = sid * ROWS_PER_TILE
    for c in range(OUT_CHUNKS):
        pltpu.sync_copy(obuf, acc.at[pl.ds(row_base + c * OUT_CHUNK, OUT_CHUNK)])

    @pl.when(sid == NUM_TILES - 1)
    def _():
        pltpu.sync_copy(obuf.at[pl.ds(0, TAIL_ROWS)],
                        acc.at[pl.ds(TAIL_BASE, TAIL_ROWS)])
    plsc.subcore_barrier()

    def e_start(c, S):
        idx_v, ridx_v, val_v, rows_v, sem_e, sem_g, sem_s = S
        base = edge_base + c * K
        pltpu.async_copy(col_hbm.at[pl.ds(base, K)], idx_v, sem_e)
        pltpu.async_copy(row_hbm.at[pl.ds(base, K)], ridx_v, sem_e)
        pltpu.async_copy(val_hbm.at[pl.ds(base, K)], val_v, sem_e)

    def e_wait(c, S):
        idx_v, ridx_v, val_v, rows_v, sem_e, sem_g, sem_s = S
        base = edge_base + c * K
        pltpu.make_async_copy(col_hbm.at[pl.ds(base, K)], idx_v, sem_e).wait()
        pltpu.make_async_copy(row_hbm.at[pl.ds(base, K)], ridx_v, sem_e).wait()
        pltpu.make_async_copy(val_hbm.at[pl.ds(base, K)], val_v, sem_e).wait()

    def g_start(S):
        idx_v, ridx_v, val_v, rows_v, sem_e, sem_g, sem_s = S
        pltpu.async_copy(feat_hbm.at[idx_v], rows_v, sem_g)

    def g_wait(S):
        idx_v, ridx_v, val_v, rows_v, sem_e, sem_g, sem_s = S
        pltpu.make_async_copy(feat_hbm.at[idx_v], rows_v, sem_g).wait()

    def s_start(S):
        idx_v, ridx_v, val_v, rows_v, sem_e, sem_g, sem_s = S
        pltpu.async_copy(rows_v, acc.at[ridx_v], sem_s, add=True)

    def s_wait(S):
        idx_v, ridx_v, val_v, rows_v, sem_e, sem_g, sem_s = S
        pltpu.make_async_copy(rows_v, acc.at[ridx_v], sem_s).wait()

    def scale(S):
        idx_v, ridx_v, val_v, rows_v, sem_e, sem_g, sem_s = S

        def e_body(t, cc):
            vv = val_v[pl.ds(t * LANES, LANES)]
            for i in range(LANES):
                e = t * LANES + i
                v = vv[i]
                for j in range(D // LANES):
                    rows_v[e, pl.ds(LANES * j, LANES)] = (
                        rows_v[e, pl.ds(LANES * j, LANES)] * v)
            return cc
        lax.fori_loop(0, K // LANES, e_body, 0)

    sets = [
        (idx0, ridx0, val0, rows0, sem_e0, sem_g0, sem_s0),
        (idx1, ridx1, val1, rows1, sem_e1, sem_g1, sem_s1),
        (idx2, ridx2, val2, rows2, sem_e2, sem_g2, sem_s2),
        (idx3, ridx3, val3, rows3, sem_e3, sem_g3, sem_s3),
    ]

    def phase(c, X, Z, drain_prev, prefetch):
        g_wait(X)
        if drain_prev:
            s_wait(Z)
        if prefetch:
            e_start(c + 3, Z)
        scale(X)
        if prefetch:
            e_wait(c + 3, Z)
            g_start(Z)
        s_start(X)

    e_start(0, sets[0])
    e_start(1, sets[1])
    e_start(2, sets[2])
    e_wait(0, sets[0])
    g_start(sets[0])
    e_wait(1, sets[1])
    g_start(sets[1])
    e_wait(2, sets[2])
    g_start(sets[2])
    phase(0, sets[0], sets[3], False, True)
    phase(1, sets[1], sets[0], True, True)
    phase(2, sets[2], sets[1], True, True)

    def quad_body(p, carry):
        c0 = 4 * p + 3
        phase(c0, sets[3], sets[2], True, True)
        phase(c0 + 1, sets[0], sets[3], True, True)
        phase(c0 + 2, sets[1], sets[0], True, True)
        phase(c0 + 3, sets[2], sets[1], True, True)
        return carry

    lax.fori_loop(0, (CHUNKS - 9) // 4, quad_body, 0)
    phase(CHUNKS - 6, sets[3], sets[2], True, True)
    phase(CHUNKS - 5, sets[0], sets[3], True, True)
    phase(CHUNKS - 4, sets[1], sets[0], True, True)
    phase(CHUNKS - 3, sets[2], sets[1], True, False)
    phase(CHUNKS - 2, sets[3], sets[2], True, False)
    phase(CHUNKS - 1, sets[0], sets[3], True, False)
    s_wait(sets[0])
    plsc.subcore_barrier()

    for c in range(OUT_CHUNKS):
        sl = pl.ds(row_base + c * OUT_CHUNK, OUT_CHUNK)
        pltpu.sync_copy(acc.at[sl], obuf)
        pltpu.sync_copy(obuf, out_hbm.at[cid].at[sl])

    @pl.when(sid == NUM_TILES - 1)
    def _():
        sl = pl.ds(TAIL_BASE, TAIL_ROWS)
        pltpu.sync_copy(acc.at[sl], obuf.at[pl.ds(0, TAIL_ROWS)])
        pltpu.sync_copy(obuf.at[pl.ds(0, TAIL_ROWS)], out_hbm.at[cid].at[sl])


def _add_body(a_ref, b_ref, o_ref):
    o_ref[...] = a_ref[...] + b_ref[...]


def kernel(adj_indices, adj_values, feature):
    row = adj_indices[0]
    col = adj_indices[1]
    mesh = plsc.VectorSubcoreMesh(
        core_axis_name="c", subcore_axis_name="s", num_cores=NUM_CORES)
    k = pl.kernel(
        _body,
        out_type=jax.ShapeDtypeStruct((NUM_CORES, N, D), jnp.float32),
        mesh=mesh,
        scratch_types=[
            pltpu.VMEM_SHARED((N, D), jnp.float32),
            pltpu.VMEM((K,), jnp.int32),
            pltpu.VMEM((K,), jnp.int32),
            pltpu.VMEM((K,), jnp.int32),
            pltpu.VMEM((K,), jnp.int32),
            pltpu.VMEM((K,), jnp.int32),
            pltpu.VMEM((K,), jnp.int32),
            pltpu.VMEM((K,), jnp.int32),
            pltpu.VMEM((K,), jnp.int32),
            pltpu.VMEM((K,), jnp.float32),
            pltpu.VMEM((K,), jnp.float32),
            pltpu.VMEM((K,), jnp.float32),
            pltpu.VMEM((K,), jnp.float32),
            pltpu.VMEM((K, D), jnp.float32),
            pltpu.VMEM((K, D), jnp.float32),
            pltpu.VMEM((K, D), jnp.float32),
            pltpu.VMEM((K, D), jnp.float32),
            pltpu.VMEM((OUT_CHUNK, D), jnp.float32),
            pltpu.SemaphoreType.DMA,
            pltpu.SemaphoreType.DMA,
            pltpu.SemaphoreType.DMA,
            pltpu.SemaphoreType.DMA,
            pltpu.SemaphoreType.DMA,
            pltpu.SemaphoreType.DMA,
            pltpu.SemaphoreType.DMA,
            pltpu.SemaphoreType.DMA,
            pltpu.SemaphoreType.DMA,
            pltpu.SemaphoreType.DMA,
            pltpu.SemaphoreType.DMA,
            pltpu.SemaphoreType.DMA,
        ],
    )
    out2 = k(row, col, adj_values, feature)

    blk = 2000
    return pl.pallas_call(
        _add_body,
        out_shape=jax.ShapeDtypeStruct((N, D), jnp.float32),
        grid=(N // blk,),
        in_specs=[pl.BlockSpec((blk, D), lambda i: (i, 0)),
                  pl.BlockSpec((blk, D), lambda i: (i, 0))],
        out_specs=pl.BlockSpec((blk, D), lambda i: (i, 0)),
    )(out2[0], out2[1])


# speedup vs baseline: 1.1253x; 1.0192x over previous
import jax
import jax.numpy as jnp
from jax import lax
from jax.experimental import pallas as pl
from jax.experimental.pallas import tpu as pltpu
from jax.experimental.pallas import tpu_sc as plsc

N = 10000
E = 320000
D = 128
LANES = 16

NUM_CORES = 2
NUM_TILES = 16
NUM_WORKERS = NUM_CORES * NUM_TILES
EPW = E // NUM_WORKERS
K = 80
CHUNKS = EPW // K
ROWS_PER_TILE = 624
OUT_CHUNK = 104
OUT_CHUNKS = ROWS_PER_TILE // OUT_CHUNK
TAIL_BASE = NUM_TILES * ROWS_PER_TILE
TAIL_ROWS = N - TAIL_BASE


def _body(row_hbm, col_hbm, val_hbm, feat_hbm, out_hbm,
          acc, idx0, idx1, idx2, ridx0, ridx1, ridx2, val0, val1, val2,
          rows0, rows1, rows2, obuf,
          sem_e0, sem_e1, sem_e2, sem_g0, sem_g1, sem_g2,
          sem_s0, sem_s1, sem_s2):
    cid = lax.axis_index("c")
    sid = lax.axis_index("s")
    wid = cid * NUM_TILES + sid
    edge_base = wid * EPW

    def zrow(r, c):
        for j in range(D // LANES):
            obuf[r, pl.ds(LANES * j, LANES)] = jnp.zeros((LANES,), jnp.float32)
        return c
    lax.fori_loop(0, OUT_CHUNK, zrow, 0)
    row_base = sid * ROWS_PER_TILE
    for c in range(OUT_CHUNKS):
        pltpu.sync_copy(obuf, acc.at[pl.ds(row_base + c * OUT_CHUNK, OUT_CHUNK)])

    @pl.when(sid == NUM_TILES - 1)
    def _():
        pltpu.sync_copy(obuf.at[pl.ds(0, TAIL_ROWS)],
                        acc.at[pl.ds(TAIL_BASE, TAIL_ROWS)])
    plsc.subcore_barrier()

    def e_start(c, S):
        idx_v, ridx_v, val_v, rows_v, sem_e, sem_g, sem_s = S
        base = edge_base + c * K
        pltpu.async_copy(col_hbm.at[pl.ds(base, K)], idx_v, sem_e)
        pltpu.async_copy(row_hbm.at[pl.ds(base, K)], ridx_v, sem_e)
        pltpu.async_copy(val_hbm.at[pl.ds(base, K)], val_v, sem_e)

    def e_wait(c, S):
        idx_v, ridx_v, val_v, rows_v, sem_e, sem_g, sem_s = S
        base = edge_base + c * K
        pltpu.make_async_copy(col_hbm.at[pl.ds(base, K)], idx_v, sem_e).wait()
        pltpu.make_async_copy(row_hbm.at[pl.ds(base, K)], ridx_v, sem_e).wait()
        pltpu.make_async_copy(val_hbm.at[pl.ds(base, K)], val_v, sem_e).wait()

    def g_start(S):
        idx_v, ridx_v, val_v, rows_v, sem_e, sem_g, sem_s = S
        pltpu.async_copy(feat_hbm.at[idx_v], rows_v, sem_g)

    def g_wait(S):
        idx_v, ridx_v, val_v, rows_v, sem_e, sem_g, sem_s = S
        pltpu.make_async_copy(feat_hbm.at[idx_v], rows_v, sem_g).wait()

    def s_start(S):
        idx_v, ridx_v, val_v, rows_v, sem_e, sem_g, sem_s = S
        pltpu.async_copy(rows_v, acc.at[ridx_v], sem_s, add=True)

    def s_wait(S):
        idx_v, ridx_v, val_v, rows_v, sem_e, sem_g, sem_s = S
        pltpu.make_async_copy(rows_v, acc.at[ridx_v], sem_s).wait()

    def scale(S):
        idx_v, ridx_v, val_v, rows_v, sem_e, sem_g, sem_s = S

        def e_body(t, cc):
            vv = val_v[pl.ds(t * LANES, LANES)]
            for i in range(LANES):
                e = t * LANES + i
                v = vv[i]
                for j in range(D // LANES):
                    rows_v[e, pl.ds(LANES * j, LANES)] = (
                        rows_v[e, pl.ds(LANES * j, LANES)] * v)
            return cc
        lax.fori_loop(0, K // LANES, e_body, 0)

    sets = [
        (idx0, ridx0, val0, rows0, sem_e0, sem_g0, sem_s0),
        (idx1, ridx1, val1, rows1, sem_e1, sem_g1, sem_s1),
        (idx2, ridx2, val2, rows2, sem_e2, sem_g2, sem_s2),
    ]

    def phase(c, X, Z, drain_prev, prefetch):
        g_wait(X)
        if drain_prev:
            s_wait(Z)
        if prefetch:
            e_start(c + 2, Z)
        scale(X)
        if prefetch:
            e_wait(c + 2, Z)
            g_start(Z)
        s_start(X)

    e_start(0, sets[0])
    e_start(1, sets[1])
    e_wait(0, sets[0])
    g_start(sets[0])
    e_wait(1, sets[1])
    g_start(sets[1])
    phase(0, sets[0], sets[2], False, True)
    phase(1, sets[1], sets[0], True, True)

    def triple_body(p, carry):
        c0 = 3 * p + 2
        phase(c0, sets[2], sets[1], True, True)
        phase(c0 + 1, sets[0], sets[2], True, True)
        phase(c0 + 2, sets[1], sets[0], True, True)
        return carry

    lax.fori_loop(0, (CHUNKS - 5) // 3, triple_body, 0)
    phase(CHUNKS - 3, sets[2], sets[1], True, True)
    phase(CHUNKS - 2, sets[0], sets[2], True, False)
    phase(CHUNKS - 1, sets[1], sets[0], True, False)
    s_wait(sets[1])
    plsc.subcore_barrier()

    for c in range(OUT_CHUNKS):
        sl = pl.ds(row_base + c * OUT_CHUNK, OUT_CHUNK)
        pltpu.sync_copy(acc.at[sl], obuf)
        pltpu.sync_copy(obuf, out_hbm.at[cid].at[sl])

    @pl.when(sid == NUM_TILES - 1)
    def _():
        sl = pl.ds(TAIL_BASE, TAIL_ROWS)
        pltpu.sync_copy(acc.at[sl], obuf.at[pl.ds(0, TAIL_ROWS)])
        pltpu.sync_copy(obuf.at[pl.ds(0, TAIL_ROWS)], out_hbm.at[cid].at[sl])


def _add_body(a_ref, b_ref, o_ref):
    o_ref[...] = a_ref[...] + b_ref[...]


def kernel(adj_indices, adj_values, feature):
    row = adj_indices[0]
    col = adj_indices[1]
    mesh = plsc.VectorSubcoreMesh(
        core_axis_name="c", subcore_axis_name="s", num_cores=NUM_CORES)
    k = pl.kernel(
        _body,
        out_type=jax.ShapeDtypeStruct((NUM_CORES, N, D), jnp.float32),
        mesh=mesh,
        scratch_types=[
            pltpu.VMEM_SHARED((N, D), jnp.float32),
            pltpu.VMEM((K,), jnp.int32),
            pltpu.VMEM((K,), jnp.int32),
            pltpu.VMEM((K,), jnp.int32),
            pltpu.VMEM((K,), jnp.int32),
            pltpu.VMEM((K,), jnp.int32),
            pltpu.VMEM((K,), jnp.int32),
            pltpu.VMEM((K,), jnp.float32),
            pltpu.VMEM((K,), jnp.float32),
            pltpu.VMEM((K,), jnp.float32),
            pltpu.VMEM((K, D), jnp.float32),
            pltpu.VMEM((K, D), jnp.float32),
            pltpu.VMEM((K, D), jnp.float32),
            pltpu.VMEM((OUT_CHUNK, D), jnp.float32),
            pltpu.SemaphoreType.DMA,
            pltpu.SemaphoreType.DMA,
            pltpu.SemaphoreType.DMA,
            pltpu.SemaphoreType.DMA,
            pltpu.SemaphoreType.DMA,
            pltpu.SemaphoreType.DMA,
            pltpu.SemaphoreType.DMA,
            pltpu.SemaphoreType.DMA,
            pltpu.SemaphoreType.DMA,
        ],
    )
    out2 = k(row, col, adj_values, feature)

    blk = 2000
    return pl.pallas_call(
        _add_body,
        out_shape=jax.ShapeDtypeStruct((N, D), jnp.float32),
        grid=(N // blk,),
        in_specs=[pl.BlockSpec((blk, D), lambda i: (i, 0)),
                  pl.BlockSpec((blk, D), lambda i: (i, 0))],
        out_specs=pl.BlockSpec((blk, D), lambda i: (i, 0)),
    )(out2[0], out2[1])


# speedup vs baseline: 1.1657x; 1.0359x over previous
import jax
import jax.numpy as jnp
from jax import lax
from jax.experimental import pallas as pl
from jax.experimental.pallas import tpu as pltpu
from jax.experimental.pallas import tpu_sc as plsc

N = 10000
E = 320000
D = 128
LANES = 16

NUM_CORES = 2
NUM_TILES = 16
NUM_WORKERS = NUM_CORES * NUM_TILES
EPW = E // NUM_WORKERS
K = 80
CHUNKS = EPW // K
ROWS_PER_TILE = 624
OUT_CHUNK = 104
OUT_CHUNKS = ROWS_PER_TILE // OUT_CHUNK
TAIL_BASE = NUM_TILES * ROWS_PER_TILE
TAIL_ROWS = N - TAIL_BASE


def _body(row_hbm, col_hbm, val_hbm, feat_hbm, out_hbm,
          acc, idx0, idx1, idx2, ridx0, ridx1, ridx2, val0, val1, val2,
          rows0, rows1, rows2, obuf,
          sem_e0, sem_e1, sem_e2, sem_g0, sem_g1, sem_g2,
          sem_s0, sem_s1, sem_s2):
    cid = lax.axis_index("c")
    sid = lax.axis_index("s")
    wid = cid * NUM_TILES + sid
    edge_base = wid * EPW

    def zrow(r, c):
        for j in range(D // LANES):
            obuf[r, pl.ds(LANES * j, LANES)] = jnp.zeros((LANES,), jnp.float32)
        return c
    lax.fori_loop(0, OUT_CHUNK, zrow, 0)
    row_base = sid * ROWS_PER_TILE
    for c in range(OUT_CHUNKS):
        pltpu.sync_copy(obuf, acc.at[pl.ds(row_base + c * OUT_CHUNK, OUT_CHUNK)])

    @pl.when(sid == NUM_TILES - 1)
    def _():
        pltpu.sync_copy(obuf.at[pl.ds(0, TAIL_ROWS)],
                        acc.at[pl.ds(TAIL_BASE, TAIL_ROWS)])
    plsc.subcore_barrier()

    def e_start(c, S):
        idx_v, ridx_v, val_v, rows_v, sem_e, sem_g, sem_s = S
        base = edge_base + c * K
        pltpu.async_copy(col_hbm.at[pl.ds(base, K)], idx_v, sem_e)
        pltpu.async_copy(row_hbm.at[pl.ds(base, K)], ridx_v, sem_e)
        pltpu.async_copy(val_hbm.at[pl.ds(base, K)], val_v, sem_e)

    def e_wait(c, S):
        idx_v, ridx_v, val_v, rows_v, sem_e, sem_g, sem_s = S
        base = edge_base + c * K
        pltpu.make_async_copy(col_hbm.at[pl.ds(base, K)], idx_v, sem_e).wait()
        pltpu.make_async_copy(row_hbm.at[pl.ds(base, K)], ridx_v, sem_e).wait()
        pltpu.make_async_copy(val_hbm.at[pl.ds(base, K)], val_v, sem_e).wait()

    def g_start(S):
        idx_v, ridx_v, val_v, rows_v, sem_e, sem_g, sem_s = S
        pltpu.async_copy(feat_hbm.at[idx_v], rows_v, sem_g)

    def g_wait(S):
        idx_v, ridx_v, val_v, rows_v, sem_e, sem_g, sem_s = S
        pltpu.make_async_copy(feat_hbm.at[idx_v], rows_v, sem_g).wait()

    def s_start(S):
        idx_v, ridx_v, val_v, rows_v, sem_e, sem_g, sem_s = S
        pltpu.async_copy(rows_v, acc.at[ridx_v], sem_s, add=True)

    def s_wait(S):
        idx_v, ridx_v, val_v, rows_v, sem_e, sem_g, sem_s = S
        pltpu.make_async_copy(rows_v, acc.at[ridx_v], sem_s).wait()

    def scale(S):
        idx_v, ridx_v, val_v, rows_v, sem_e, sem_g, sem_s = S

        def e_body(t, cc):
            vv = val_v[pl.ds(t * LANES, LANES)]
            for i in range(LANES):
                e = t * LANES + i
                v = vv[i]
                for j in range(D // LANES):
                    rows_v[e, pl.ds(LANES * j, LANES)] = (
                        rows_v[e, pl.ds(LANES * j, LANES)] * v)
            return cc
        lax.fori_loop(0, K // LANES, e_body, 0)

    sets = [
        (idx0, ridx0, val0, rows0, sem_e0, sem_g0, sem_s0),
        (idx1, ridx1, val1, rows1, sem_e1, sem_g1, sem_s1),
        (idx2, ridx2, val2, rows2, sem_e2, sem_g2, sem_s2),
    ]

    def phase(c, X, Z, drain_prev, prefetch):
        g_wait(X)
        if drain_prev:
            s_wait(Z)
        if prefetch:
            e_start(c + 2, Z)
        scale(X)
        if prefetch:
            e_wait(c + 2, Z)
            g_start(Z)
        s_start(X)

    e_start(0, sets[0])
    e_start(1, sets[1])
    e_wait(0, sets[0])
    g_start(sets[0])
    e_wait(1, sets[1])
    g_start(sets[1])
    phase(0, sets[0], sets[2], False, True)
    phase(1, sets[1], sets[0], True, True)

    def triple_body(p, carry):
        c0 = 3 * p + 2
        phase(c0, sets[2], sets[1], True, True)
        phase(c0 + 1, sets[0], sets[2], True, True)
        phase(c0 + 2, sets[1], sets[0], True, True)
        return carry

    lax.fori_loop(0, (CHUNKS - 5) // 3, triple_body, 0)
    phase(CHUNKS - 3, sets[2], sets[1], True, True)
    phase(CHUNKS - 2, sets[0], sets[2], True, False)
    phase(CHUNKS - 1, sets[1], sets[0], True, False)
    s_wait(sets[1])
    plsc.subcore_barrier()

    for c in range(OUT_CHUNKS):
        sl = pl.ds(row_base + c * OUT_CHUNK, OUT_CHUNK)
        pltpu.sync_copy(acc.at[sl], obuf)
        pltpu.sync_copy(obuf, out_hbm.at[cid].at[sl])

    @pl.when(sid == NUM_TILES - 1)
    def _():
        sl = pl.ds(TAIL_BASE, TAIL_ROWS)
        pltpu.sync_copy(acc.at[sl], obuf.at[pl.ds(0, TAIL_ROWS)])
        pltpu.sync_copy(obuf.at[pl.ds(0, TAIL_ROWS)], out_hbm.at[cid].at[sl])


def _add_body(a_ref, b_ref, o_ref):
    o_ref[...] = a_ref[0] + b_ref[0]


def kernel(adj_indices, adj_values, feature):
    row = adj_indices[0]
    col = adj_indices[1]
    mesh = plsc.VectorSubcoreMesh(
        core_axis_name="c", subcore_axis_name="s", num_cores=NUM_CORES)
    k = pl.kernel(
        _body,
        out_type=jax.ShapeDtypeStruct((NUM_CORES, N, D), jnp.float32),
        mesh=mesh,
        scratch_types=[
            pltpu.VMEM_SHARED((N, D), jnp.float32),
            pltpu.VMEM((K,), jnp.int32),
            pltpu.VMEM((K,), jnp.int32),
            pltpu.VMEM((K,), jnp.int32),
            pltpu.VMEM((K,), jnp.int32),
            pltpu.VMEM((K,), jnp.int32),
            pltpu.VMEM((K,), jnp.int32),
            pltpu.VMEM((K,), jnp.float32),
            pltpu.VMEM((K,), jnp.float32),
            pltpu.VMEM((K,), jnp.float32),
            pltpu.VMEM((K, D), jnp.float32),
            pltpu.VMEM((K, D), jnp.float32),
            pltpu.VMEM((K, D), jnp.float32),
            pltpu.VMEM((OUT_CHUNK, D), jnp.float32),
            pltpu.SemaphoreType.DMA,
            pltpu.SemaphoreType.DMA,
            pltpu.SemaphoreType.DMA,
            pltpu.SemaphoreType.DMA,
            pltpu.SemaphoreType.DMA,
            pltpu.SemaphoreType.DMA,
            pltpu.SemaphoreType.DMA,
            pltpu.SemaphoreType.DMA,
            pltpu.SemaphoreType.DMA,
        ],
    )
    out2 = k(row, col, adj_values, feature)

    blk = 2000
    return pl.pallas_call(
        _add_body,
        out_shape=jax.ShapeDtypeStruct((N, D), jnp.float32),
        grid=(N // blk,),
        in_specs=[pl.BlockSpec((1, blk, D), lambda i: (0, i, 0)),
                  pl.BlockSpec((1, blk, D), lambda i: (1, i, 0))],
        out_specs=pl.BlockSpec((blk, D), lambda i: (i, 0)),
    )(out2, out2)


# speedup vs baseline: 1.1701x; 1.0037x over previous
import jax
import jax.numpy as jnp
from jax import lax
from jax.experimental import pallas as pl
from jax.experimental.pallas import tpu as pltpu
from jax.experimental.pallas import tpu_sc as plsc

N = 10000
E = 320000
D = 128
LANES = 16

NUM_CORES = 2
NUM_TILES = 16
NUM_WORKERS = NUM_CORES * NUM_TILES
EPW = E // NUM_WORKERS
K = 80
CHUNKS = EPW // K
ROWS_PER_TILE = 624
OUT_CHUNK = 104
OUT_CHUNKS = ROWS_PER_TILE // OUT_CHUNK
TAIL_BASE = NUM_TILES * ROWS_PER_TILE
TAIL_ROWS = N - TAIL_BASE


def _body(row_hbm, col_hbm, val_hbm, feat_hbm, out_hbm,
          acc, idx0, idx1, idx2, ridx0, ridx1, ridx2, val0, val1, val2,
          rows0, rows1, rows2, obuf,
          sem_e0, sem_e1, sem_e2, sem_g0, sem_g1, sem_g2,
          sem_s0, sem_s1, sem_s2):
    cid = lax.axis_index("c")
    sid = lax.axis_index("s")
    wid = cid * NUM_TILES + sid
    edge_base = wid * EPW

    def zrow(r, c):
        for j in range(D // LANES):
            obuf[r, pl.ds(LANES * j, LANES)] = jnp.zeros((LANES,), jnp.float32)
        return c
    lax.fori_loop(0, OUT_CHUNK, zrow, 0)
    row_base = sid * ROWS_PER_TILE
    for c in range(OUT_CHUNKS):
        pltpu.async_copy(obuf, acc.at[pl.ds(row_base + c * OUT_CHUNK, OUT_CHUNK)],
                         sem_e0)

    @pl.when(sid == NUM_TILES - 1)
    def _():
        pltpu.async_copy(obuf.at[pl.ds(0, TAIL_ROWS)],
                         acc.at[pl.ds(TAIL_BASE, TAIL_ROWS)], sem_e1)
    for c in range(OUT_CHUNKS):
        pltpu.make_async_copy(
            obuf, acc.at[pl.ds(row_base + c * OUT_CHUNK, OUT_CHUNK)],
            sem_e0).wait()

    @pl.when(sid == NUM_TILES - 1)
    def _():
        pltpu.make_async_copy(obuf.at[pl.ds(0, TAIL_ROWS)],
                              acc.at[pl.ds(TAIL_BASE, TAIL_ROWS)],
                              sem_e1).wait()
    plsc.subcore_barrier()

    def e_start(c, S):
        idx_v, ridx_v, val_v, rows_v, sem_e, sem_g, sem_s = S
        base = edge_base + c * K
        pltpu.async_copy(col_hbm.at[pl.ds(base, K)], idx_v, sem_e)
        pltpu.async_copy(row_hbm.at[pl.ds(base, K)], ridx_v, sem_e)
        pltpu.async_copy(val_hbm.at[pl.ds(base, K)], val_v, sem_e)

    def e_wait(c, S):
        idx_v, ridx_v, val_v, rows_v, sem_e, sem_g, sem_s = S
        base = edge_base + c * K
        pltpu.make_async_copy(col_hbm.at[pl.ds(base, K)], idx_v, sem_e).wait()
        pltpu.make_async_copy(row_hbm.at[pl.ds(base, K)], ridx_v, sem_e).wait()
        pltpu.make_async_copy(val_hbm.at[pl.ds(base, K)], val_v, sem_e).wait()

    def g_start(S):
        idx_v, ridx_v, val_v, rows_v, sem_e, sem_g, sem_s = S
        pltpu.async_copy(feat_hbm.at[idx_v], rows_v, sem_g)

    def g_wait(S):
        idx_v, ridx_v, val_v, rows_v, sem_e, sem_g, sem_s = S
        pltpu.make_async_copy(feat_hbm.at[idx_v], rows_v, sem_g).wait()

    def s_start(S):
        idx_v, ridx_v, val_v, rows_v, sem_e, sem_g, sem_s = S
        pltpu.async_copy(rows_v, acc.at[ridx_v], sem_s, add=True)

    def s_wait(S):
        idx_v, ridx_v, val_v, rows_v, sem_e, sem_g, sem_s = S
        pltpu.make_async_copy(rows_v, acc.at[ridx_v], sem_s).wait()

    def scale(S):
        idx_v, ridx_v, val_v, rows_v, sem_e, sem_g, sem_s = S

        def e_body(t, cc):
            vv = val_v[pl.ds(t * LANES, LANES)]
            for i in range(LANES):
                e = t * LANES + i
                v = vv[i]
                for j in range(D // LANES):
                    rows_v[e, pl.ds(LANES * j, LANES)] = (
                        rows_v[e, pl.ds(LANES * j, LANES)] * v)
            return cc
        lax.fori_loop(0, K // LANES, e_body, 0)

    sets = [
        (idx0, ridx0, val0, rows0, sem_e0, sem_g0, sem_s0),
        (idx1, ridx1, val1, rows1, sem_e1, sem_g1, sem_s1),
        (idx2, ridx2, val2, rows2, sem_e2, sem_g2, sem_s2),
    ]

    def phase(c, X, Z, drain_prev, prefetch):
        g_wait(X)
        if drain_prev:
            s_wait(Z)
        if prefetch:
            e_start(c + 2, Z)
        scale(X)
        if prefetch:
            e_wait(c + 2, Z)
            g_start(Z)
        s_start(X)

    e_start(0, sets[0])
    e_start(1, sets[1])
    e_wait(0, sets[0])
    g_start(sets[0])
    e_wait(1, sets[1])
    g_start(sets[1])
    phase(0, sets[0], sets[2], False, True)
    phase(1, sets[1], sets[0], True, True)

    def triple_body(p, carry):
        c0 = 3 * p + 2
        phase(c0, sets[2], sets[1], True, True)
        phase(c0 + 1, sets[0], sets[2], True, True)
        phase(c0 + 2, sets[1], sets[0], True, True)
        return carry

    lax.fori_loop(0, (CHUNKS - 5) // 3, triple_body, 0)
    phase(CHUNKS - 3, sets[2], sets[1], True, True)
    phase(CHUNKS - 2, sets[0], sets[2], True, False)
    phase(CHUNKS - 1, sets[1], sets[0], True, False)
    s_wait(sets[1])
    plsc.subcore_barrier()

    osl = pl.ds(row_base, ROWS_PER_TILE)
    pltpu.async_copy(acc.at[osl], out_hbm.at[cid].at[osl], sem_g0)

    @pl.when(sid == NUM_TILES - 1)
    def _():
        tsl = pl.ds(TAIL_BASE, TAIL_ROWS)
        pltpu.async_copy(acc.at[tsl], out_hbm.at[cid].at[tsl], sem_g1)
    pltpu.make_async_copy(acc.at[osl], out_hbm.at[cid].at[osl], sem_g0).wait()

    @pl.when(sid == NUM_TILES - 1)
    def _():
        tsl = pl.ds(TAIL_BASE, TAIL_ROWS)
        pltpu.make_async_copy(acc.at[tsl], out_hbm.at[cid].at[tsl],
                              sem_g1).wait()


def _add_body(a_ref, b_ref, o_ref):
    o_ref[...] = a_ref[0] + b_ref[0]


def kernel(adj_indices, adj_values, feature):
    row = adj_indices[0]
    col = adj_indices[1]
    mesh = plsc.VectorSubcoreMesh(
        core_axis_name="c", subcore_axis_name="s", num_cores=NUM_CORES)
    k = pl.kernel(
        _body,
        out_type=jax.ShapeDtypeStruct((NUM_CORES, N, D), jnp.float32),
        mesh=mesh,
        scratch_types=[
            pltpu.VMEM_SHARED((N, D), jnp.float32),
            pltpu.VMEM((K,), jnp.int32),
            pltpu.VMEM((K,), jnp.int32),
            pltpu.VMEM((K,), jnp.int32),
            pltpu.VMEM((K,), jnp.int32),
            pltpu.VMEM((K,), jnp.int32),
            pltpu.VMEM((K,), jnp.int32),
            pltpu.VMEM((K,), jnp.float32),
            pltpu.VMEM((K,), jnp.float32),
            pltpu.VMEM((K,), jnp.float32),
            pltpu.VMEM((K, D), jnp.float32),
            pltpu.VMEM((K, D), jnp.float32),
            pltpu.VMEM((K, D), jnp.float32),
            pltpu.VMEM((OUT_CHUNK, D), jnp.float32),
            pltpu.SemaphoreType.DMA,
            pltpu.SemaphoreType.DMA,
            pltpu.SemaphoreType.DMA,
            pltpu.SemaphoreType.DMA,
            pltpu.SemaphoreType.DMA,
            pltpu.SemaphoreType.DMA,
            pltpu.SemaphoreType.DMA,
            pltpu.SemaphoreType.DMA,
            pltpu.SemaphoreType.DMA,
        ],
    )
    out2 = k(row, col, adj_values, feature)

    blk = 2000
    return pl.pallas_call(
        _add_body,
        out_shape=jax.ShapeDtypeStruct((N, D), jnp.float32),
        grid=(N // blk,),
        in_specs=[pl.BlockSpec((1, blk, D), lambda i: (0, i, 0)),
                  pl.BlockSpec((1, blk, D), lambda i: (1, i, 0))],
        out_specs=pl.BlockSpec((blk, D), lambda i: (i, 0)),
    )(out2, out2)
